# baseline (speedup 1.0000x reference)
import functools

import jax
import jax.numpy as jnp
from jax import lax
from jax.experimental import pallas as pl

_EPS = 1e-3


def _segsum_kernel(idx_ref, x_ref, y_ref, oxy_ref, ox_ref, os_ref, *, W, R, CH):
    i = pl.program_id(0)

    @pl.when(i == 0)
    def _init():
        oxy_ref[...] = jnp.zeros_like(oxy_ref)
        ox_ref[...] = jnp.zeros_like(ox_ref)
        os_ref[...] = jnp.zeros_like(os_ref)

    for c in range(R // CH):
        idx = idx_ref[0, 0, c * CH:(c + 1) * CH]
        x = x_ref[c * CH:(c + 1) * CH, :]
        y = y_ref[c * CH:(c + 1) * CH, :]

        f_xy = x * y
        ssq = jnp.sum(x * x, axis=1, keepdims=True)
        li = lax.broadcasted_iota(jnp.int32, (CH, 128), 1)
        f_s = (
            jnp.where(li == 0, y, 0.0)
            + jnp.where(li == 1, ssq, 0.0)
            + jnp.where(li == 2, 1.0, 0.0)
        )

        idx_b = jnp.broadcast_to(idx[None, :], (W, CH))
        w_iota = lax.broadcasted_iota(jnp.int32, (W, CH), 0)

        w0 = jnp.min(idx) // W
        w1 = jnp.max(idx) // W

        f_xy_b = f_xy.astype(jnp.bfloat16)
        x_b = x.astype(jnp.bfloat16)
        f_s_b = f_s.astype(jnp.bfloat16)

        def body(w, _, idx_b=idx_b, w_iota=w_iota, f_xy=f_xy_b, x=x_b, f_s=f_s_b):
            onehot = (idx_b == (w_iota + w * W)).astype(jnp.bfloat16)
            dn = (((1,), (0,)), ((), ()))
            pxy = lax.dot_general(onehot, f_xy, dn, preferred_element_type=jnp.float32)
            px = lax.dot_general(onehot, x, dn, preferred_element_type=jnp.float32)
            ps = lax.dot_general(onehot, f_s, dn, preferred_element_type=jnp.float32)
            sl = pl.ds(w * W, W)
            oxy_ref[sl, :] += pxy
            ox_ref[sl, :] += px
            os_ref[sl, :] += ps
            return 0

        lax.fori_loop(w0, w1 + 1, body, 0)


def _head_kernel(sxy_ref, sx_ref, ss_ref, ax_ref, ay_ref, w1_ref, b1_ref,
                 w2_ref, b2_ref, rep_ref, mix_ref, scale_ref, a_ref, b_ref,
                 *, D, K):
    sxy = sxy_ref[...]
    sx = sx_ref[...]
    ss = ss_ref[...]
    sy = ss[:, 0:1]
    sxx = ss[:, 1:2]
    n = ss[:, 2:3]
    ax = ax_ref[...]
    ay = ay_ref[...]

    rep = sxy - ay * sx - sy * ax + n * (ax * ay)

    counts = n * D
    seg_sum = jnp.sum(sx, axis=1, keepdims=True) - n * jnp.sum(
        ax, axis=1, keepdims=True)
    seg_sumsq = (sxx - 2.0 * jnp.sum(ax * sx, axis=1, keepdims=True)
                 + n * jnp.sum(ax * ax, axis=1, keepdims=True))
    mean = seg_sum / jnp.maximum(counts, 1.0)
    var = (seg_sumsq - counts * mean * mean) / jnp.maximum(counts - 1.0, 1.0)
    rep = (1.0 / var) * rep
    rep_ref[...] = rep

    dn = (((1,), (0,)), ((), ()))
    h = jnp.tanh(
        lax.dot_general(rep, w1_ref[...], dn, preferred_element_type=jnp.float32)
        + b1_ref[...])
    out = lax.dot_general(h, w2_ref[...], dn,
                          preferred_element_type=jnp.float32) + b2_ref[...]

    def softplus(v):
        return jnp.maximum(v, 0.0) + jnp.log1p(jnp.exp(-jnp.abs(v)))

    alpha = softplus(out[:, 0:1]) * (1.0 - _EPS) + _EPS
    beta = softplus(out[:, 1:2]) * (1.0 - _EPS) + _EPS
    a_ref[...] = alpha
    b_ref[...] = beta
    scale_ref[...] = jnp.sqrt(beta / alpha)

    mix = out[:, 2:2 + K]
    m = jnp.max(mix, axis=1, keepdims=True)
    e = jnp.exp(mix - m)
    mix_ref[...] = e / jnp.sum(e, axis=1, keepdims=True)


def kernel(index, x, y, anchor_x, anchor_y, W1, b1, W2, b2):
    N, D = x.shape
    S = anchor_x.shape[0]
    H = W1.shape[1]
    K = W2.shape[1] - 2

    if N % 8000 == 0:
        R, CH = 8000, 800
    else:
        R = CH = 512
    W = 64
    assert N % R == 0 and R % CH == 0 and D == 128
    NB = N // R
    S_pad = ((S + W - 1) // W) * W

    idx3 = index.reshape(NB, 1, R)

    seg_fn = pl.pallas_call(
        functools.partial(_segsum_kernel, W=W, R=R, CH=CH),
        grid=(NB,),
        in_specs=[
            pl.BlockSpec((1, 1, R), lambda i: (i, 0, 0)),
            pl.BlockSpec((R, D), lambda i: (i, 0)),
            pl.BlockSpec((R, 1), lambda i: (i, 0)),
        ],
        out_specs=[
            pl.BlockSpec((S_pad, D), lambda i: (0, 0)),
            pl.BlockSpec((S_pad, D), lambda i: (0, 0)),
            pl.BlockSpec((S_pad, 128), lambda i: (0, 0)),
        ],
        out_shape=[
            jax.ShapeDtypeStruct((S_pad, D), jnp.float32),
            jax.ShapeDtypeStruct((S_pad, D), jnp.float32),
            jax.ShapeDtypeStruct((S_pad, 128), jnp.float32),
        ],
    )
    sxy, sx, ss = seg_fn(idx3, x, y)

    SB = 1000
    assert S % SB == 0
    NS = S // SB

    head_fn = pl.pallas_call(
        functools.partial(_head_kernel, D=float(D), K=K),
        grid=(NS,),
        in_specs=[
            pl.BlockSpec((SB, D), lambda i: (i, 0)),
            pl.BlockSpec((SB, D), lambda i: (i, 0)),
            pl.BlockSpec((SB, 128), lambda i: (i, 0)),
            pl.BlockSpec((SB, D), lambda i: (i, 0)),
            pl.BlockSpec((SB, 1), lambda i: (i, 0)),
            pl.BlockSpec((D, H), lambda i: (0, 0)),
            pl.BlockSpec((1, H), lambda i: (0, 0)),
            pl.BlockSpec((H, K + 2), lambda i: (0, 0)),
            pl.BlockSpec((1, K + 2), lambda i: (0, 0)),
        ],
        out_specs=[
            pl.BlockSpec((SB, D), lambda i: (i, 0)),
            pl.BlockSpec((SB, K), lambda i: (i, 0)),
            pl.BlockSpec((SB, 1), lambda i: (i, 0)),
            pl.BlockSpec((SB, 1), lambda i: (i, 0)),
            pl.BlockSpec((SB, 1), lambda i: (i, 0)),
        ],
        out_shape=[
            jax.ShapeDtypeStruct((S, D), jnp.float32),
            jax.ShapeDtypeStruct((S, K), jnp.float32),
            jax.ShapeDtypeStruct((S, 1), jnp.float32),
            jax.ShapeDtypeStruct((S, 1), jnp.float32),
            jax.ShapeDtypeStruct((S, 1), jnp.float32),
        ],
    )
    rep, mixture, scale, alpha, beta = head_fn(
        sxy, sx, ss, anchor_x, anchor_y, W1,
        b1.reshape(1, H), W2, b2.reshape(1, K + 2))
    return rep, mixture, scale, alpha, beta


# speedup vs baseline: 1.2030x; 1.2030x over previous
import functools

import jax
import jax.numpy as jnp
from jax import lax
from jax.experimental import pallas as pl

_EPS = 1e-3


def _segsum_kernel(idx_ref, x_ref, y_ref, oxy_ref, ox_ref, oq_ref, os_ref,
                   *, W, R, CH):
    i = pl.program_id(0)

    @pl.when(i == 0)
    def _init():
        oxy_ref[...] = jnp.zeros_like(oxy_ref)
        ox_ref[...] = jnp.zeros_like(ox_ref)
        oq_ref[...] = jnp.zeros_like(oq_ref)
        os_ref[...] = jnp.zeros_like(os_ref)

    for c in range(R // CH):
        idx = idx_ref[0, 0, c * CH:(c + 1) * CH]
        x = x_ref[c * CH:(c + 1) * CH, :]
        y = y_ref[c * CH:(c + 1) * CH, :]

        f_xy = x * y
        f_sq = x * x
        li = lax.broadcasted_iota(jnp.int32, (CH, 128), 1)
        f_s = jnp.where(li == 0, y, 0.0) + jnp.where(li == 1, 1.0, 0.0)

        idx_b = jnp.broadcast_to(idx[None, :], (W, CH))
        w_iota = lax.broadcasted_iota(jnp.int32, (W, CH), 0)

        w0 = jnp.min(idx) // W
        w1 = jnp.max(idx) // W

        def body(w, _, idx_b=idx_b, w_iota=w_iota, f_xy=f_xy, x=x,
                 f_sq=f_sq, f_s=f_s):
            onehot = (idx_b == (w_iota + w * W)).astype(jnp.float32)
            dn = (((1,), (0,)), ((), ()))
            pxy = lax.dot_general(onehot, f_xy, dn, preferred_element_type=jnp.float32)
            px = lax.dot_general(onehot, x, dn, preferred_element_type=jnp.float32)
            pq = lax.dot_general(onehot, f_sq, dn, preferred_element_type=jnp.float32)
            ps = lax.dot_general(onehot, f_s, dn, preferred_element_type=jnp.float32)
            sl = pl.ds(w * W, W)
            oxy_ref[sl, :] += pxy
            ox_ref[sl, :] += px
            oq_ref[sl, :] += pq
            os_ref[sl, :] += ps
            return 0

        lax.fori_loop(w0, w1 + 1, body, 0)


def _head_kernel(sxy_ref, sx_ref, sq_ref, ss_ref, ax_ref, ay_ref, w1_ref,
                 b1_ref, w2_ref, b2_ref, rep_ref, mix_ref, scale_ref, a_ref,
                 b_ref, *, D, K):
    sxy = sxy_ref[...]
    sx = sx_ref[...]
    ss = ss_ref[...]
    sy = ss[:, 0:1]
    sxx = jnp.sum(sq_ref[...], axis=1, keepdims=True)
    n = ss[:, 1:2]
    ax = ax_ref[...]
    ay = ay_ref[...]

    rep = sxy - ay * sx - sy * ax + n * (ax * ay)

    counts = n * D
    seg_sum = jnp.sum(sx, axis=1, keepdims=True) - n * jnp.sum(
        ax, axis=1, keepdims=True)
    seg_sumsq = (sxx - 2.0 * jnp.sum(ax * sx, axis=1, keepdims=True)
                 + n * jnp.sum(ax * ax, axis=1, keepdims=True))
    mean = seg_sum / jnp.maximum(counts, 1.0)
    var = (seg_sumsq - counts * mean * mean) / jnp.maximum(counts - 1.0, 1.0)
    rep = (1.0 / var) * rep
    rep_ref[...] = rep

    dn = (((1,), (0,)), ((), ()))
    h = jnp.tanh(
        lax.dot_general(rep, w1_ref[...], dn, preferred_element_type=jnp.float32)
        + b1_ref[...])
    out = lax.dot_general(h, w2_ref[...], dn,
                          preferred_element_type=jnp.float32) + b2_ref[...]

    def softplus(v):
        return jnp.maximum(v, 0.0) + jnp.log1p(jnp.exp(-jnp.abs(v)))

    alpha = softplus(out[:, 0:1]) * (1.0 - _EPS) + _EPS
    beta = softplus(out[:, 1:2]) * (1.0 - _EPS) + _EPS
    a_ref[...] = alpha
    b_ref[...] = beta
    scale_ref[...] = jnp.sqrt(beta / alpha)

    mix = out[:, 2:2 + K]
    m = jnp.max(mix, axis=1, keepdims=True)
    e = jnp.exp(mix - m)
    mix_ref[...] = e / jnp.sum(e, axis=1, keepdims=True)


def kernel(index, x, y, anchor_x, anchor_y, W1, b1, W2, b2):
    N, D = x.shape
    S = anchor_x.shape[0]
    H = W1.shape[1]
    K = W2.shape[1] - 2

    if N % 8000 == 0:
        R, CH = 8000, 1600
    else:
        R = CH = 512
    W = 64
    assert N % R == 0 and R % CH == 0 and D == 128
    NB = N // R
    S_pad = ((S + W - 1) // W) * W

    idx3 = index.reshape(NB, 1, R)

    seg_fn = pl.pallas_call(
        functools.partial(_segsum_kernel, W=W, R=R, CH=CH),
        grid=(NB,),
        in_specs=[
            pl.BlockSpec((1, 1, R), lambda i: (i, 0, 0)),
            pl.BlockSpec((R, D), lambda i: (i, 0)),
            pl.BlockSpec((R, 1), lambda i: (i, 0)),
        ],
        out_specs=[
            pl.BlockSpec((S_pad, D), lambda i: (0, 0)),
            pl.BlockSpec((S_pad, D), lambda i: (0, 0)),
            pl.BlockSpec((S_pad, D), lambda i: (0, 0)),
            pl.BlockSpec((S_pad, 128), lambda i: (0, 0)),
        ],
        out_shape=[
            jax.ShapeDtypeStruct((S_pad, D), jnp.float32),
            jax.ShapeDtypeStruct((S_pad, D), jnp.float32),
            jax.ShapeDtypeStruct((S_pad, D), jnp.float32),
            jax.ShapeDtypeStruct((S_pad, 128), jnp.float32),
        ],
    )
    sxy, sx, sq, ss = seg_fn(idx3, x, y)

    SB = 1000
    assert S % SB == 0
    NS = S // SB

    head_fn = pl.pallas_call(
        functools.partial(_head_kernel, D=float(D), K=K),
        grid=(NS,),
        in_specs=[
            pl.BlockSpec((SB, D), lambda i: (i, 0)),
            pl.BlockSpec((SB, D), lambda i: (i, 0)),
            pl.BlockSpec((SB, D), lambda i: (i, 0)),
            pl.BlockSpec((SB, 128), lambda i: (i, 0)),
            pl.BlockSpec((SB, D), lambda i: (i, 0)),
            pl.BlockSpec((SB, 1), lambda i: (i, 0)),
            pl.BlockSpec((D, H), lambda i: (0, 0)),
            pl.BlockSpec((1, H), lambda i: (0, 0)),
            pl.BlockSpec((H, K + 2), lambda i: (0, 0)),
            pl.BlockSpec((1, K + 2), lambda i: (0, 0)),
        ],
        out_specs=[
            pl.BlockSpec((SB, D), lambda i: (i, 0)),
            pl.BlockSpec((SB, K), lambda i: (i, 0)),
            pl.BlockSpec((SB, 1), lambda i: (i, 0)),
            pl.BlockSpec((SB, 1), lambda i: (i, 0)),
            pl.BlockSpec((SB, 1), lambda i: (i, 0)),
        ],
        out_shape=[
            jax.ShapeDtypeStruct((S, D), jnp.float32),
            jax.ShapeDtypeStruct((S, K), jnp.float32),
            jax.ShapeDtypeStruct((S, 1), jnp.float32),
            jax.ShapeDtypeStruct((S, 1), jnp.float32),
            jax.ShapeDtypeStruct((S, 1), jnp.float32),
        ],
    )
    rep, mixture, scale, alpha, beta = head_fn(
        sxy, sx, sq, ss, anchor_x, anchor_y, W1,
        b1.reshape(1, H), W2, b2.reshape(1, K + 2))
    return rep, mixture, scale, alpha, beta
